# Initial kernel scaffold; baseline (speedup 1.0000x reference)
import jax
import jax.numpy as jnp
from jax.experimental import pallas as pl


def kernel(dense_inputs, sparse_inputs, w0, w, v):
    raise NotImplementedError("write your pallas kernel here")


# speedup vs baseline: 23.7981x; 23.7981x over previous
import functools

import jax
import jax.numpy as jnp
from jax import lax
from jax.experimental import pallas as pl
from jax.experimental.pallas import tpu as pltpu
from jax.experimental.pallas import tpu_sc as plsc

N_DENSE = 13
N_SPARSE = 26
K = 16
FIELD_NUM = 39
D = FIELD_NUM * K
IDX_PAD = 32

NC, NS = 2, 16
NW = NC * NS


GRP = 2
GIDX = 56


def _sc_gather(table, idxg, idxw, wcol, batch):
  b_per_w = batch // NW
  g_per_w = b_per_w // GRP
  feat = table.shape[0]
  mesh = plsc.VectorSubcoreMesh(
      core_axis_name="c", subcore_axis_name="s", num_cores=NC, num_subcores=NS)

  @functools.partial(
      pl.kernel,
      out_type=[
          jax.ShapeDtypeStruct((batch, D), jnp.float32),
          jax.ShapeDtypeStruct((batch, IDX_PAD), jnp.float32),
      ],
      mesh=mesh,
      compiler_params=pltpu.CompilerParams(
          needs_layout_passes=False, use_tc_tiling_on_sc=False),
      scratch_types=[
          pltpu.VMEM((g_per_w, GIDX), jnp.int32),
          pltpu.VMEM((b_per_w, IDX_PAD), jnp.int32),
          pltpu.VMEM((feat,), jnp.float32),
          pltpu.VMEM((GIDX, D), jnp.float32),
          pltpu.VMEM((GIDX, D), jnp.float32),
          pltpu.VMEM((2, GRP, D), jnp.float32),
          pltpu.VMEM((b_per_w, IDX_PAD), jnp.float32),
          pltpu.SemaphoreType.DMA,
          pltpu.SemaphoreType.DMA,
          pltpu.SemaphoreType.DMA,
          pltpu.SemaphoreType.DMA,
      ],
  )
  def k(table_h, idxg_h, idxw_h, w_h, g_h, gw_h, idx_v, idxw_v, wtab,
        buf0, buf1, ost, gwst, gsem0, gsem1, wsem0, wsem1):
    wid = lax.axis_index("s") * NC + lax.axis_index("c")
    base = wid * b_per_w
    gbase = wid * g_per_w
    pltpu.sync_copy(idxg_h.at[pl.ds(gbase, g_per_w)], idx_v)
    pltpu.sync_copy(idxw_h.at[pl.ds(base, b_per_w)], idxw_v)
    pltpu.sync_copy(w_h, wtab)

    bufs = (buf0, buf1)
    gsems = (gsem0, gsem1)
    wsems = (wsem0, wsem1)
    lane = lax.iota(jnp.int32, 16)
    tailmask = jnp.where(lane < N_SPARSE - 16, 1.0, 0.0)

    def gather_desc(grp, par):
      src = table_h.at[idx_v.at[grp, pl.ds(0, GIDX)]]
      return pltpu.make_async_copy(src, bufs[par], gsems[par])

    def write_desc(grp, par):
      return pltpu.make_async_copy(
          ost.at[par], g_h.at[pl.ds(base + GRP * grp, GRP)], wsems[par])

    gather_desc(0, 0).start()
    gather_desc(1, 1).start()

    def half(i, par):
      grp = 2 * i + par
      gather_desc(grp, par).wait()
      for j in range(GRP):
        s = GRP * grp + j
        iv0 = idxw_v[s, pl.ds(0, 16)]
        iv1 = idxw_v[s, pl.ds(16, 16)]
        gwst[s, pl.ds(0, 16)] = plsc.load_gather(wtab, [iv0])
        gwst[s, pl.ds(16, 16)] = plsc.load_gather(wtab, [iv1]) * tailmask
      @pl.when(i > 0)
      def _():
        write_desc(grp - 2, par).wait()
      buf = bufs[par]

      def col(c):
        sl = pl.ds(c * 16, 16)
        for j in range(GRP):
          off = j * N_SPARSE
          acc = buf[off, sl]
          for f in range(1, N_SPARSE):
            acc = acc + buf[off + f, sl]
          ost[par, j, sl] = acc

      pl.loop(0, FIELD_NUM)(col)
      write_desc(grp, par).start()
      @pl.when(grp + 2 < g_per_w)
      def _():
        gather_desc(grp + 2, par).start()

    def body(i):
      half(i, 0)
      half(i, 1)

    pl.loop(0, g_per_w // 2)(body)

    write_desc(g_per_w - 2, 0).wait()
    write_desc(g_per_w - 1, 1).wait()
    pltpu.sync_copy(gwst, gw_h.at[pl.ds(base, b_per_w)])

  return k(table, idxg, idxw, wcol)


def _tc_finalize(dense, g, gw, v13, w13, w0c, batch):
  blk = 512

  def body(dense_ref, g_ref, gw_ref, v13_ref, w13_ref, w0_ref, o_ref):
    f = jnp.dot(dense_ref[...], v13_ref[...],
                preferred_element_type=jnp.float32) + g_ref[...]
    p = jnp.sum(f * f, axis=1, keepdims=True)
    col = lax.broadcasted_iota(jnp.int32, (D, K), 0)
    row = lax.broadcasted_iota(jnp.int32, (D, K), 1)
    a = jnp.where(col % K == row, 1.0, 0.0)
    s = jnp.dot(f, a, preferred_element_type=jnp.float32)
    s2 = jnp.sum(s * s, axis=1, keepdims=True)
    first = (w0_ref[0, 0]
             + jnp.sum(dense_ref[...] * w13_ref[...], axis=1, keepdims=True)
             + jnp.sum(gw_ref[...], axis=1, keepdims=True))
    o_ref[...] = jax.nn.sigmoid(first + 0.5 * (s2 - p))

  return pl.pallas_call(
      body,
      grid=(batch // blk,),
      in_specs=[
          pl.BlockSpec((blk, N_DENSE), lambda i: (i, 0)),
          pl.BlockSpec((blk, D), lambda i: (i, 0)),
          pl.BlockSpec((blk, IDX_PAD), lambda i: (i, 0)),
          pl.BlockSpec((N_DENSE, D), lambda i: (0, 0)),
          pl.BlockSpec((1, N_DENSE), lambda i: (0, 0)),
          pl.BlockSpec((1, 1), lambda i: (0, 0)),
      ],
      out_specs=pl.BlockSpec((blk, 1), lambda i: (i, 0)),
      out_shape=jax.ShapeDtypeStruct((batch, 1), jnp.float32),
  )(dense, g, gw, v13, w13, w0c)


def kernel(dense_inputs, sparse_inputs, w0, w, v):
  batch = dense_inputs.shape[0]
  feat, field_num, k = v.shape
  d = field_num * k
  assert d == D and k == K

  vocab = (feat - N_DENSE) // N_SPARSE
  offsets = jnp.arange(N_SPARSE, dtype=jnp.int32) * vocab + N_DENSE
  idx = sparse_inputs + offsets[None, :]
  idxw = jnp.concatenate(
      [idx, jnp.zeros((batch, IDX_PAD - N_SPARSE), jnp.int32)], axis=1)
  idx2 = idx.reshape(batch // GRP, GRP * N_SPARSE)
  idxg = jnp.concatenate(
      [idx2, jnp.zeros((batch // GRP, GIDX - GRP * N_SPARSE), jnp.int32)],
      axis=1)

  v_flat = v.reshape(feat, d)
  g, gw = _sc_gather(v_flat, idxg, idxw, w[:, 0], batch)
  return _tc_finalize(dense_inputs, g, gw, v_flat[:N_DENSE],
                      w[:N_DENSE].reshape(1, N_DENSE),
                      w0.reshape(1, 1), batch)
